# Initial kernel scaffold; baseline (speedup 1.0000x reference)
import jax
import jax.numpy as jnp
from jax import lax
from jax.experimental import pallas as pl
from jax.experimental.pallas import tpu as pltpu
from jax.experimental.pallas import tpu_sc as plsc

N = 10000
D_IN = 128
D_HID = 256
E = 320000

NC, NS = 2, 16
NW = NC * NS
CHUNK = 128
CH = (E + NW * CHUNK - 1) // (NW * CHUNK)
E_PAD = NW * CH * CHUNK
N_PAD = 10240
ROWS_PER_SUB = N_PAD // NS


def _sc_segment_sum(with_cnt):
    mesh = plsc.VectorSubcoreMesh(core_axis_name="c", subcore_axis_name="s")
    out_type = [jax.ShapeDtypeStruct((NC, N_PAD, D_IN), jnp.float32)]
    if with_cnt:
        out_type.append(jax.ShapeDtypeStruct((NC, N_PAD, 16), jnp.float32))
    scratch = [
        pltpu.VMEM((CHUNK,), jnp.int32),
        pltpu.VMEM((CHUNK,), jnp.int32),
        pltpu.VMEM((CHUNK, D_IN), jnp.float32),
        pltpu.VMEM((16, D_IN), jnp.float32),
        pltpu.VMEM_SHARED((N_PAD, D_IN), jnp.float32),
    ]
    if with_cnt:
        scratch.append(pltpu.VMEM((CHUNK, 16), jnp.float32))
        scratch.append(pltpu.VMEM_SHARED((N_PAD, 16), jnp.float32))

    def body(x_hbm, src_hbm, dst_hbm, out_hbm, *rest):
        if with_cnt:
            cnt_hbm, src_v, dst_v, rows_v, zero_v, acc_sh, ones_v, cnt_sh = rest
        else:
            src_v, dst_v, rows_v, zero_v, acc_sh = rest
        c = lax.axis_index("c")
        s = lax.axis_index("s")
        w = c * NS + s

        zrow = jnp.zeros((16,), jnp.float32)
        for i in range(16):
            for j in range(D_IN // 16):
                zero_v[i, pl.ds(j * 16, 16)] = zrow

        base = s * ROWS_PER_SUB

        @pl.loop(0, ROWS_PER_SUB // 16)
        def _(k):
            pltpu.sync_copy(zero_v, acc_sh.at[pl.ds(base + k * 16, 16)])

        if with_cnt:
            one = jnp.ones((16,), jnp.float32)

            @pl.loop(0, CHUNK)
            def _(i):
                ones_v[i, pl.ds(0, 16)] = one

            @pl.loop(0, ROWS_PER_SUB // 16)
            def _(k):
                pltpu.sync_copy(zero_v.at[pl.ds(0, 16), pl.ds(0, 16)],
                                cnt_sh.at[pl.ds(base + k * 16, 16)])

        plsc.subcore_barrier()

        edge_base = w * CH * CHUNK

        @pl.loop(0, CH)
        def _(i):
            off = edge_base + i * CHUNK
            pltpu.sync_copy(src_hbm.at[pl.ds(off, CHUNK)], src_v)
            pltpu.sync_copy(dst_hbm.at[pl.ds(off, CHUNK)], dst_v)
            pltpu.sync_copy(x_hbm.at[src_v], rows_v)
            pltpu.sync_copy(rows_v, acc_sh.at[dst_v], add=True)
            if with_cnt:
                pltpu.sync_copy(ones_v, cnt_sh.at[dst_v], add=True)

        plsc.subcore_barrier()

        pltpu.sync_copy(acc_sh.at[pl.ds(base, ROWS_PER_SUB)],
                        out_hbm.at[c, pl.ds(base, ROWS_PER_SUB)])
        if with_cnt:
            pltpu.sync_copy(cnt_sh.at[pl.ds(base, ROWS_PER_SUB)],
                            cnt_hbm.at[c, pl.ds(base, ROWS_PER_SUB)])

    return pl.kernel(body, out_type=out_type, mesh=mesh, scratch_types=scratch)


_sc_sum_cnt = _sc_segment_sum(with_cnt=True)
_sc_sum = _sc_segment_sum(with_cnt=False)

_BLK = 1000


def _mm1_body(aggp_ref, cntp_ref, x_ref, w1l_ref, b1l_ref, w1r_ref,
              w2l_ref, b2l_ref, w2r_ref, z_ref, r2_ref, invb_ref):
    agg = aggp_ref[0] + aggp_ref[1]
    cnt = cntp_ref[0, :, :1] + cntp_ref[1, :, :1]
    inv = 1.0 / jnp.maximum(cnt, 1.0)
    aggm = agg * inv
    dn = (((1,), (1,)), ((), ()))
    h = lax.dot_general(aggm, w1l_ref[...], dn,
                        preferred_element_type=jnp.float32)
    h = h + b1l_ref[...][None, :]
    h = h + lax.dot_general(x_ref[...], w1r_ref[...], dn,
                            preferred_element_type=jnp.float32)
    h = jnp.maximum(h, 0.0)
    z_ref[...] = lax.dot_general(h, w2l_ref[...], dn,
                                 preferred_element_type=jnp.float32)
    r2_ref[...] = lax.dot_general(h, w2r_ref[...], dn,
                                  preferred_element_type=jnp.float32) \
        + b2l_ref[...][None, :]
    invb_ref[...] = jnp.broadcast_to(inv, (inv.shape[0], D_IN))


def _tc_mm1(aggp, cntp, x, w1l, b1l, w1r, w2l, b2l, w2r):
    nb = N // _BLK
    full = lambda shape: pl.BlockSpec(shape, lambda i: (0,) * len(shape))
    return pl.pallas_call(
        _mm1_body,
        grid=(nb,),
        in_specs=[
            pl.BlockSpec((NC, _BLK, D_IN), lambda i: (0, i, 0)),
            pl.BlockSpec((NC, _BLK, 16), lambda i: (0, i, 0)),
            pl.BlockSpec((_BLK, D_IN), lambda i: (i, 0)),
            full((D_HID, D_IN)), full((D_HID,)), full((D_HID, D_IN)),
            full((D_IN, D_HID)), full((D_IN,)), full((D_IN, D_HID)),
        ],
        out_specs=[
            pl.BlockSpec((_BLK, D_IN), lambda i: (i, 0)),
            pl.BlockSpec((_BLK, D_IN), lambda i: (i, 0)),
            pl.BlockSpec((_BLK, D_IN), lambda i: (i, 0)),
        ],
        out_shape=[jax.ShapeDtypeStruct((N, D_IN), jnp.float32)] * 3,
    )(aggp, cntp, x, w1l, b1l, w1r, w2l, b2l, w2r)


def _final_body(aggzp_ref, invb_ref, r2_ref, out_ref):
    out_ref[...] = (aggzp_ref[0] + aggzp_ref[1]) * invb_ref[...] + r2_ref[...]


def _tc_final(aggzp, invb, r2):
    nb = N // _BLK
    return pl.pallas_call(
        _final_body,
        grid=(nb,),
        in_specs=[
            pl.BlockSpec((NC, _BLK, D_IN), lambda i: (0, i, 0)),
            pl.BlockSpec((_BLK, D_IN), lambda i: (i, 0)),
            pl.BlockSpec((_BLK, D_IN), lambda i: (i, 0)),
        ],
        out_specs=pl.BlockSpec((_BLK, D_IN), lambda i: (i, 0)),
        out_shape=jax.ShapeDtypeStruct((N, D_IN), jnp.float32),
    )(aggzp, invb, r2)


@jax.jit
def kernel(x, edge_index, W1_l, b1_l, W1_r, W2_l, b2_l, W2_r):
    src = edge_index[0].astype(jnp.int32)
    dst = edge_index[1].astype(jnp.int32)
    pad = E_PAD - E
    src_p = jnp.concatenate([src, jnp.zeros((pad,), jnp.int32)])
    dst_p = jnp.concatenate([dst, jnp.full((pad,), N, jnp.int32)])

    agg1p, cntp = _sc_sum_cnt(x, src_p, dst_p)
    z, r2, invb = _tc_mm1(agg1p[:, :N], cntp[:, :N], x,
                          W1_l, b1_l, W1_r, W2_l, b2_l, W2_r)
    aggzp = _sc_sum(z, src_p, dst_p)
    return _tc_final(aggzp[:, :N], invb, r2)


# speedup vs baseline: 4.9016x; 4.9016x over previous
import dataclasses

import jax
import jax.numpy as jnp
from jax import lax
from jax.experimental import pallas as pl
from jax.experimental.pallas import tpu as pltpu
from jax.experimental.pallas import tpu_sc as plsc

N = 10000
D_IN = 128
D_HID = 256
E = 320000

NC, NS = 2, 16
NW = NC * NS
L = 16
CHUNK = 128
CH = (E + NW * CHUNK - 1) // (NW * CHUNK)
E_PAD = NW * CH * CHUNK
N_PAD = 10240
CNT_ROWS = N_PAD // 128
ROWS_PER_SUB = N_PAD // NS
CROWS_PER_SUB = 8
CNT_REDUCERS = CNT_ROWS // CROWS_PER_SUB


def _sc_segment_sum(with_cnt):
    mesh = plsc.VectorSubcoreMesh(core_axis_name="c", subcore_axis_name="s")
    out_type = [jax.ShapeDtypeStruct((NC, N_PAD, D_IN), jnp.float32)]
    if with_cnt:
        out_type.append(jax.ShapeDtypeStruct((NC, CNT_ROWS, 128), jnp.float32))
    scratch = [
        pltpu.VMEM((CHUNK,), jnp.int32),
        pltpu.VMEM((CHUNK,), jnp.int32),
        pltpu.VMEM((CHUNK, D_IN), jnp.float32),
        pltpu.VMEM((16, D_IN), jnp.float32),
        pltpu.VMEM_SHARED((N_PAD, D_IN), jnp.float32),
    ]
    if with_cnt:
        scratch += [
            pltpu.VMEM((CNT_ROWS, 128), jnp.float32),
            pltpu.VMEM((CROWS_PER_SUB, 128), jnp.float32),
            pltpu.VMEM((CROWS_PER_SUB, 128), jnp.float32),
            pltpu.VMEM_SHARED((NS, CNT_ROWS, 128), jnp.float32),
        ]

    def body(x_hbm, src_hbm, dst_hbm, out_hbm, *rest):
        if with_cnt:
            (cnt_hbm, src_v, dst_v, rows_v, zero_v, acc_sh,
             cnt_v, tmp_v, racc_v, stage_sh) = rest
        else:
            src_v, dst_v, rows_v, zero_v, acc_sh = rest
        c = lax.axis_index("c")
        s = lax.axis_index("s")
        w = c * NS + s

        zrow = jnp.zeros((L,), jnp.float32)
        for i in range(16):
            for j in range(D_IN // L):
                zero_v[i, pl.ds(j * L, L)] = zrow

        base = s * ROWS_PER_SUB

        @pl.loop(0, ROWS_PER_SUB // 16)
        def _(k):
            pltpu.sync_copy(zero_v, acc_sh.at[pl.ds(base + k * 16, 16)])

        if with_cnt:
            @pl.loop(0, CNT_ROWS)
            def _(r):
                for j in range(128 // L):
                    cnt_v[r, pl.ds(j * L, L)] = zrow

        plsc.subcore_barrier()

        edge_base = w * CH * CHUNK
        ones = jnp.ones((L,), jnp.float32)

        @pl.loop(0, CH)
        def _(i):
            off = edge_base + i * CHUNK
            pltpu.sync_copy(src_hbm.at[pl.ds(off, CHUNK)], src_v)
            pltpu.sync_copy(dst_hbm.at[pl.ds(off, CHUNK)], dst_v)
            pltpu.sync_copy(x_hbm.at[src_v], rows_v)
            pltpu.sync_copy(rows_v, acc_sh.at[dst_v], add=True)
            if with_cnt:
                for g in range(CHUNK // L):
                    d16 = dst_v[pl.ds(g * L, L)]
                    row = lax.shift_right_logical(d16, 7)
                    col = jnp.bitwise_and(d16, 127)
                    plsc.addupdate_scatter(cnt_v, [row, col], ones)

        if with_cnt:
            pltpu.sync_copy(cnt_v, stage_sh.at[s])

        plsc.subcore_barrier()

        pltpu.sync_copy(acc_sh.at[pl.ds(base, ROWS_PER_SUB)],
                        out_hbm.at[c, pl.ds(base, ROWS_PER_SUB)])

        if with_cnt:
            @pl.when(s < CNT_REDUCERS)
            def _():
                cbase = s * CROWS_PER_SUB
                pltpu.sync_copy(stage_sh.at[0, pl.ds(cbase, CROWS_PER_SUB)],
                                racc_v)

                @pl.loop(1, NS)
                def _(k):
                    pltpu.sync_copy(
                        stage_sh.at[k, pl.ds(cbase, CROWS_PER_SUB)], tmp_v)
                    for r in range(CROWS_PER_SUB):
                        for j in range(128 // L):
                            sl = pl.ds(j * L, L)
                            racc_v[r, sl] = racc_v[r, sl] + tmp_v[r, sl]

                pltpu.sync_copy(racc_v,
                                cnt_hbm.at[c, pl.ds(cbase, CROWS_PER_SUB)])

    cp = pltpu.CompilerParams()
    if "needs_layout_passes" in pltpu.CompilerParams.__dataclass_fields__:
        cp = dataclasses.replace(cp, needs_layout_passes=False)
    return pl.kernel(body, out_type=out_type, mesh=mesh, scratch_types=scratch,
                     compiler_params=cp)


_sc_sum_cnt = _sc_segment_sum(with_cnt=True)
_sc_sum = _sc_segment_sum(with_cnt=False)

_BLK = 128


def _mm1_body(aggp_ref, cntp_ref, x_ref, w1l_ref, b1l_ref, w1r_ref,
              w2l_ref, b2l_ref, w2r_ref, z_ref, r2_ref, invb_ref):
    agg = aggp_ref[0] + aggp_ref[1]
    cnt = cntp_ref[0, 0] + cntp_ref[1, 0]
    inv = 1.0 / jnp.maximum(cnt, 1.0)
    invb = lax.dot_general(inv, jnp.ones((1, D_IN), jnp.float32),
                           (((0,), (0,)), ((), ())),
                           preferred_element_type=jnp.float32)
    aggm = agg * invb
    dn = (((1,), (1,)), ((), ()))
    h = lax.dot_general(aggm, w1l_ref[...], dn,
                        preferred_element_type=jnp.float32)
    h = h + b1l_ref[...][None, :]
    h = h + lax.dot_general(x_ref[...], w1r_ref[...], dn,
                            preferred_element_type=jnp.float32)
    h = jnp.maximum(h, 0.0)
    z_ref[...] = lax.dot_general(h, w2l_ref[...], dn,
                                 preferred_element_type=jnp.float32)
    r2_ref[...] = lax.dot_general(h, w2r_ref[...], dn,
                                  preferred_element_type=jnp.float32) \
        + b2l_ref[...][None, :]
    invb_ref[...] = invb


def _tc_mm1(aggp, cntp, x, w1l, b1l, w1r, w2l, b2l, w2r):
    nb = N_PAD // _BLK
    full = lambda shape: pl.BlockSpec(shape, lambda i: (0,) * len(shape))
    return pl.pallas_call(
        _mm1_body,
        grid=(nb,),
        in_specs=[
            pl.BlockSpec((NC, _BLK, D_IN), lambda i: (0, i, 0)),
            pl.BlockSpec((NC, 1, 1, 128), lambda i: (0, i, 0, 0)),
            pl.BlockSpec((_BLK, D_IN), lambda i: (i, 0)),
            full((D_HID, D_IN)), full((D_HID,)), full((D_HID, D_IN)),
            full((D_IN, D_HID)), full((D_IN,)), full((D_IN, D_HID)),
        ],
        out_specs=[
            pl.BlockSpec((_BLK, D_IN), lambda i: (i, 0)),
            pl.BlockSpec((_BLK, D_IN), lambda i: (i, 0)),
            pl.BlockSpec((_BLK, D_IN), lambda i: (i, 0)),
        ],
        out_shape=[jax.ShapeDtypeStruct((N_PAD, D_IN), jnp.float32)] * 3,
    )(aggp, cntp, x, w1l, b1l, w1r, w2l, b2l, w2r)


def _final_body(aggzp_ref, invb_ref, r2_ref, out_ref):
    out_ref[...] = (aggzp_ref[0] + aggzp_ref[1]) * invb_ref[...] + r2_ref[...]


def _tc_final(aggzp, invb, r2):
    nb = N_PAD // _BLK
    return pl.pallas_call(
        _final_body,
        grid=(nb,),
        in_specs=[
            pl.BlockSpec((NC, _BLK, D_IN), lambda i: (0, i, 0)),
            pl.BlockSpec((_BLK, D_IN), lambda i: (i, 0)),
            pl.BlockSpec((_BLK, D_IN), lambda i: (i, 0)),
        ],
        out_specs=pl.BlockSpec((_BLK, D_IN), lambda i: (i, 0)),
        out_shape=jax.ShapeDtypeStruct((N_PAD, D_IN), jnp.float32),
    )(aggzp, invb, r2)


@jax.jit
def kernel(x, edge_index, W1_l, b1_l, W1_r, W2_l, b2_l, W2_r):
    src = edge_index[0].astype(jnp.int32)
    dst = edge_index[1].astype(jnp.int32)
    pad = E_PAD - E
    src_p = jnp.concatenate([src, jnp.zeros((pad,), jnp.int32)])
    dst_p = jnp.concatenate([dst, jnp.full((pad,), N, jnp.int32)])
    x_p = jnp.pad(x, ((0, N_PAD - N), (0, 0)))

    agg1p, cntp = _sc_sum_cnt(x_p, src_p, dst_p)
    cntp = cntp.reshape(NC, CNT_ROWS, 1, 128)
    z, r2, invb = _tc_mm1(agg1p, cntp, x_p,
                          W1_l, b1_l, W1_r, W2_l, b2_l, W2_r)
    (aggzp,) = _sc_sum(z, src_p, dst_p)
    out = _tc_final(aggzp, invb, r2)
    return out[:N]
